# Initial kernel scaffold; baseline (speedup 1.0000x reference)
import jax
import jax.numpy as jnp
from jax.experimental import pallas as pl


def kernel(node_feature, adj, curvatures, W1, b1, W2, b2):
    raise NotImplementedError("write your pallas kernel here")


# speedup vs baseline: 2.0095x; 2.0095x over previous
import functools

import jax
import jax.numpy as jnp
from jax.experimental import pallas as pl
from jax.experimental.pallas import tpu as pltpu


def _pick_bm(n):
    for bm in range(512, 0, -8):
        if n % bm == 0:
            return bm
    return n


def _pass1_body(adj_ref, x_ref, wf_ref, c1_ref, s2_ref, t_ref):
    @pl.when(pl.program_id(0) == 0)
    def _compute_t():
        t = jnp.dot(
            x_ref[...].astype(jnp.bfloat16),
            wf_ref[...].astype(jnp.bfloat16),
            preferred_element_type=jnp.float32,
        )
        t_ref[...] = t.astype(jnp.bfloat16)

    a = adj_ref[...].astype(jnp.bfloat16)
    s2 = jnp.dot(a, t_ref[...], preferred_element_type=jnp.float32)
    s2_ref[...] = (s2 + c1_ref[...]).astype(jnp.bfloat16)


def _pass2_body(adj_ref, s2_ref, b2_ref, out_ref, *, num_manifold, d_emb):
    a = adj_ref[...].astype(jnp.bfloat16)
    o = jnp.dot(a, s2_ref[...], preferred_element_type=jnp.float32)
    o = o + b2_ref[...]
    for i in range(num_manifold):
        out_ref[i, :, :] = o[:, i * d_emb:(i + 1) * d_emb]


def kernel(node_feature, adj, curvatures, W1, b1, W2, b2):
    del curvatures
    n = adj.shape[0]
    num_manifold, d_feat, d_emb = W1.shape[0], W1.shape[1], W2.shape[2]
    c = num_manifold * d_emb
    bm = _pick_bm(n)
    nb = n // bm

    wf = jnp.concatenate([W1[i] @ W2[i] for i in range(num_manifold)], axis=1)
    c1 = jnp.concatenate([b1[i] @ W2[i] for i in range(num_manifold)])[None, :]
    b2c = jnp.concatenate([b2[i] for i in range(num_manifold)])[None, :]

    s2 = pl.pallas_call(
        _pass1_body,
        grid=(nb,),
        in_specs=[
            pl.BlockSpec((bm, n), lambda m: (m, 0)),
            pl.BlockSpec((n, d_feat), lambda m: (0, 0)),
            pl.BlockSpec((d_feat, c), lambda m: (0, 0)),
            pl.BlockSpec((1, c), lambda m: (0, 0)),
        ],
        out_specs=pl.BlockSpec((bm, c), lambda m: (m, 0)),
        out_shape=jax.ShapeDtypeStruct((n, c), jnp.bfloat16),
        scratch_shapes=[pltpu.VMEM((n, c), jnp.bfloat16)],
        compiler_params=pltpu.CompilerParams(
            dimension_semantics=("arbitrary",),
        ),
    )(adj, node_feature, wf, c1)

    out = pl.pallas_call(
        functools.partial(_pass2_body, num_manifold=num_manifold, d_emb=d_emb),
        grid=(nb,),
        in_specs=[
            pl.BlockSpec((bm, n), lambda m: (m, 0)),
            pl.BlockSpec((n, c), lambda m: (0, 0)),
            pl.BlockSpec((1, c), lambda m: (0, 0)),
        ],
        out_specs=pl.BlockSpec((num_manifold, bm, d_emb), lambda m: (0, m, 0)),
        out_shape=jax.ShapeDtypeStruct((num_manifold, n, d_emb), jnp.float32),
        compiler_params=pltpu.CompilerParams(
            dimension_semantics=("arbitrary",),
        ),
    )(adj, s2, b2c)
    return out
